# Initial kernel scaffold; baseline (speedup 1.0000x reference)
import jax
import jax.numpy as jnp
from jax.experimental import pallas as pl


def kernel(x, r1_w, r1_b, r2_w, r2_b, dw0_w, dw0_b, pw0_w, pw0_b, dw1_w, dw1_b, pw1_w, pw1_b, dw2_w, dw2_b, pw2_w, pw2_b, bn_gamma, bn_beta):
    raise NotImplementedError("write your pallas kernel here")


# speedup vs baseline: 1.0282x; 1.0282x over previous
import functools

import jax
import jax.numpy as jnp
from jax.experimental import pallas as pl

_C = 96
_KS = (3, 5, 7)
_HT = 8
_PAD = 3


def _silu(v):
    return v * jax.nn.sigmoid(v)


def _pass1_body(xc_ref, xn_ref,
                r1w_ref, r1b_ref, r2w_ref, r2b_ref,
                dw0_ref, db0_ref, pw0_ref, pb0_ref,
                dw1_ref, db1_ref, pw1_ref, pb1_ref,
                dw2_ref, db2_ref, pw2_ref, pb2_ref,
                out_ref, s1_ref, s2_ref):
    HT = out_ref.shape[1]
    W = out_ref.shape[2]
    C = out_ref.shape[3]
    a = jnp.concatenate([xc_ref[0], xn_ref[0, :2 * _PAD]], axis=0)
    xcen = a[_PAD:_PAD + HT, _PAD:_PAD + W, :].reshape(HT * W, C)

    r = jnp.dot(xcen, r1w_ref[...], preferred_element_type=jnp.float32)
    r = _silu(r + r1b_ref[...])
    logits = jnp.dot(r, r2w_ref[...], preferred_element_type=jnp.float32)
    logits = logits + r2b_ref[...]
    m = jnp.max(logits, axis=1, keepdims=True)
    p = jnp.exp(logits - m)
    rw = p / jnp.sum(p, axis=1, keepdims=True)

    out = jnp.zeros((HT * W, C), jnp.float32)
    experts = ((dw0_ref, db0_ref, pw0_ref, pb0_ref),
               (dw1_ref, db1_ref, pw1_ref, pb1_ref),
               (dw2_ref, db2_ref, pw2_ref, pb2_ref))
    for e, k in enumerate(_KS):
        dwr, dbr, pwr, pbr = experts[e]
        off = _PAD - k // 2
        acc = jnp.zeros((HT, W, C), jnp.float32)
        for i in range(k):
            for j in range(k):
                tap = dwr[i * k + j, :][None, None, :]
                acc = acc + a[off + i:off + i + HT, off + j:off + j + W, :] * tap
        y = _silu(acc + dbr[...][None]).reshape(HT * W, C)
        eo = jnp.dot(y, pwr[...], preferred_element_type=jnp.float32)
        eo = eo + pbr[...]
        out = out + eo * rw[:, e:e + 1]

    out_ref[0] = out.reshape(HT, W, C)
    s1_ref[0, 0] = jnp.sum(out, axis=0, keepdims=True)
    s2_ref[0, 0] = jnp.sum(out * out, axis=0, keepdims=True)


def _pass2_body(out_ref, s1_ref, s2_ref, g_ref, b_ref, y_ref, *, n):
    s1 = jnp.sum(s1_ref[...], axis=(0, 1, 2))
    s2 = jnp.sum(s2_ref[...], axis=(0, 1, 2))
    mean = s1 / n
    var = s2 / n - mean * mean
    scale = g_ref[0] * jax.lax.rsqrt(var + 1e-5)
    shift = b_ref[0] - mean * scale
    y = out_ref[0] * scale[None, None, :] + shift[None, None, :]
    y_ref[0] = _silu(y)


def kernel(x, r1_w, r1_b, r2_w, r2_b,
           dw0_w, dw0_b, pw0_w, pw0_b,
           dw1_w, dw1_b, pw1_w, pw1_b,
           dw2_w, dw2_b, pw2_w, pw2_b,
           bn_gamma, bn_beta):
    B, C, H, W = x.shape
    HT = _HT
    T = H // HT
    HP = (T + 1) * HT
    Wp = W + 2 * _PAD

    xt = jnp.transpose(x, (0, 2, 3, 1))
    xp = jnp.pad(xt, ((0, 0), (_PAD, HP - H - _PAD), (_PAD, _PAD), (0, 0)))

    wargs = (
        r1_w.T, r1_b[None], r2_w.T, r2_b[None],
        dw0_w.reshape(C, -1).T, dw0_b[None], pw0_w.T, pw0_b[None],
        dw1_w.reshape(C, -1).T, dw1_b[None], pw1_w.T, pw1_b[None],
        dw2_w.reshape(C, -1).T, dw2_b[None], pw2_w.T, pw2_b[None],
    )

    def full_spec(arr):
        nd = arr.ndim
        return pl.BlockSpec(arr.shape, lambda b, t, _nd=nd: (0,) * _nd)

    xblk = pl.BlockSpec((1, HT, Wp, C), lambda b, t: (b, t, 0, 0))
    xblk_next = pl.BlockSpec((1, HT, Wp, C), lambda b, t: (b, t + 1, 0, 0))

    out, s1, s2 = pl.pallas_call(
        _pass1_body,
        out_shape=(
            jax.ShapeDtypeStruct((B, H, W, C), jnp.float32),
            jax.ShapeDtypeStruct((B, T, 1, C), jnp.float32),
            jax.ShapeDtypeStruct((B, T, 1, C), jnp.float32),
        ),
        grid=(B, T),
        in_specs=[xblk, xblk_next] + [full_spec(w) for w in wargs],
        out_specs=(
            pl.BlockSpec((1, HT, W, C), lambda b, t: (b, t, 0, 0)),
            pl.BlockSpec((1, 1, 1, C), lambda b, t: (b, t, 0, 0)),
            pl.BlockSpec((1, 1, 1, C), lambda b, t: (b, t, 0, 0)),
        ),
    )(xp, xp, *wargs)

    n = float(B * H * W)
    y = pl.pallas_call(
        functools.partial(_pass2_body, n=n),
        out_shape=jax.ShapeDtypeStruct((B, H, W, C), jnp.float32),
        grid=(B, T),
        in_specs=[
            pl.BlockSpec((1, HT, W, C), lambda b, t: (b, t, 0, 0)),
            full_spec(s1),
            full_spec(s2),
            pl.BlockSpec((1, C), lambda b, t: (0, 0)),
            pl.BlockSpec((1, C), lambda b, t: (0, 0)),
        ],
        out_specs=pl.BlockSpec((1, HT, W, C), lambda b, t: (b, t, 0, 0)),
    )(out, s1, s2, bn_gamma[None], bn_beta[None])

    return jnp.transpose(y, (0, 3, 1, 2))


# speedup vs baseline: 1.3926x; 1.3543x over previous
import functools

import jax
import jax.numpy as jnp
from jax.experimental import pallas as pl
from jax.experimental.pallas import tpu as pltpu

_C = 96
_KS = (3, 5, 7)
_HT = 8
_PAD = 3


def _silu(v):
    return v * jax.nn.sigmoid(v)


def _pass1_body(xc_ref, xn_ref,
                r1w_ref, r1b_ref, r2w_ref, r2b_ref,
                dw0_ref, db0_ref, pw0_ref, pb0_ref,
                dw1_ref, db1_ref, pw1_ref, pb1_ref,
                dw2_ref, db2_ref, pw2_ref, pb2_ref,
                out_ref, s1_ref, s2_ref, slab_ref):
    HT = out_ref.shape[1]
    W = out_ref.shape[2]
    C = out_ref.shape[3]
    a = jnp.concatenate([xc_ref[0], xn_ref[0, :2 * _PAD]], axis=0)
    xcen = a[_PAD:_PAD + HT, _PAD:_PAD + W, :].reshape(HT * W, C)

    r = jnp.dot(xcen, r1w_ref[...], preferred_element_type=jnp.float32)
    r = _silu(r + r1b_ref[...])
    logits = jnp.dot(r, r2w_ref[...], preferred_element_type=jnp.float32)
    logits = logits + r2b_ref[...]
    m = jnp.max(logits, axis=1, keepdims=True)
    p = jnp.exp(logits - m)
    rw = p / jnp.sum(p, axis=1, keepdims=True)

    for j in range(2 * _PAD + 1):
        slab_ref[j] = a[:, j:j + W, :]

    out = jnp.zeros((HT * W, C), jnp.float32)
    experts = ((dw0_ref, db0_ref, pw0_ref, pb0_ref),
               (dw1_ref, db1_ref, pw1_ref, pb1_ref),
               (dw2_ref, db2_ref, pw2_ref, pb2_ref))
    for e, k in enumerate(_KS):
        dwr, dbr, pwr, pbr = experts[e]
        off = _PAD - k // 2
        acc = jnp.zeros((HT, W, C), jnp.float32)
        for i in range(k):
            for j in range(k):
                tap = dwr[i * k + j, :][None, None, :]
                acc = acc + slab_ref[off + j, off + i:off + i + HT] * tap
        y = _silu(acc + dbr[...][None]).reshape(HT * W, C)
        eo = jnp.dot(y, pwr[...], preferred_element_type=jnp.float32)
        eo = eo + pbr[...]
        out = out + eo * rw[:, e:e + 1]

    out_ref[0] = out.reshape(HT, W, C)
    s1_ref[0, 0] = jnp.sum(out, axis=0, keepdims=True)
    s2_ref[0, 0] = jnp.sum(out * out, axis=0, keepdims=True)


def _pass2_body(out_ref, s1_ref, s2_ref, g_ref, b_ref, y_ref, *, n):
    s1 = jnp.sum(s1_ref[...], axis=(0, 1, 2))
    s2 = jnp.sum(s2_ref[...], axis=(0, 1, 2))
    mean = s1 / n
    var = s2 / n - mean * mean
    scale = g_ref[0] * jax.lax.rsqrt(var + 1e-5)
    shift = b_ref[0] - mean * scale
    y = out_ref[0] * scale[None, None, :] + shift[None, None, :]
    y_ref[0] = _silu(y)


def kernel(x, r1_w, r1_b, r2_w, r2_b,
           dw0_w, dw0_b, pw0_w, pw0_b,
           dw1_w, dw1_b, pw1_w, pw1_b,
           dw2_w, dw2_b, pw2_w, pw2_b,
           bn_gamma, bn_beta):
    B, C, H, W = x.shape
    HT = _HT
    T = H // HT
    HP = (T + 1) * HT
    Wp = W + 2 * _PAD

    xt = jnp.transpose(x, (0, 2, 3, 1))
    xp = jnp.pad(xt, ((0, 0), (_PAD, HP - H - _PAD), (_PAD, _PAD), (0, 0)))

    wargs = (
        r1_w.T, r1_b[None], r2_w.T, r2_b[None],
        dw0_w.reshape(C, -1).T, dw0_b[None], pw0_w.T, pw0_b[None],
        dw1_w.reshape(C, -1).T, dw1_b[None], pw1_w.T, pw1_b[None],
        dw2_w.reshape(C, -1).T, dw2_b[None], pw2_w.T, pw2_b[None],
    )

    def full_spec(arr):
        nd = arr.ndim
        return pl.BlockSpec(arr.shape, lambda b, t, _nd=nd: (0,) * _nd)

    xblk = pl.BlockSpec((1, HT, Wp, C), lambda b, t: (b, t, 0, 0))
    xblk_next = pl.BlockSpec((1, HT, Wp, C), lambda b, t: (b, t + 1, 0, 0))

    out, s1, s2 = pl.pallas_call(
        _pass1_body,
        out_shape=(
            jax.ShapeDtypeStruct((B, H, W, C), jnp.float32),
            jax.ShapeDtypeStruct((B, T, 1, C), jnp.float32),
            jax.ShapeDtypeStruct((B, T, 1, C), jnp.float32),
        ),
        grid=(B, T),
        in_specs=[xblk, xblk_next] + [full_spec(w) for w in wargs],
        out_specs=(
            pl.BlockSpec((1, HT, W, C), lambda b, t: (b, t, 0, 0)),
            pl.BlockSpec((1, 1, 1, C), lambda b, t: (b, t, 0, 0)),
            pl.BlockSpec((1, 1, 1, C), lambda b, t: (b, t, 0, 0)),
        ),
        scratch_shapes=[
            pltpu.VMEM((2 * _PAD + 1, HT + 2 * _PAD, W, C), jnp.float32),
        ],
    )(xp, xp, *wargs)

    n = float(B * H * W)
    y = pl.pallas_call(
        functools.partial(_pass2_body, n=n),
        out_shape=jax.ShapeDtypeStruct((B, H, W, C), jnp.float32),
        grid=(B, T),
        in_specs=[
            pl.BlockSpec((1, HT, W, C), lambda b, t: (b, t, 0, 0)),
            full_spec(s1),
            full_spec(s2),
            pl.BlockSpec((1, C), lambda b, t: (0, 0)),
            pl.BlockSpec((1, C), lambda b, t: (0, 0)),
        ],
        out_specs=pl.BlockSpec((1, HT, W, C), lambda b, t: (b, t, 0, 0)),
    )(out, s1, s2, bn_gamma[None], bn_beta[None])

    return jnp.transpose(y, (0, 3, 1, 2))
